# baseline (speedup 1.0000x reference)
import jax
import jax.numpy as jnp
from jax import lax
from jax.experimental import pallas as pl

B = 1024
C = 100000
P = 50
EPS = 1e-07

R = 32
PPAD = 64
NT_FULL = C // 128


def _loss_body(yt_ref, ypred_ref, out_ref):
    x = ypred_ref[...]
    m = jnp.max(x, axis=1, keepdims=True)
    m0 = jnp.maximum(m, 0.0)
    s = jnp.sum(jnp.exp(x - m0), axis=1, keepdims=True)
    all_loss = m0 + jnp.log(s + jnp.exp(-m0))

    yt = yt_ref[...]
    tk = yt >> 7
    lk = yt & 127

    def tile_body(t, acc):
        xt = ypred_ref[:, pl.ds(t * 128, 128)]
        g = jnp.take_along_axis(xt, lk, axis=1)
        return jnp.where(tk == t, g, acc)

    acc = lax.fori_loop(0, NT_FULL, tile_body,
                        jnp.zeros((R, PPAD), jnp.float32))
    xt = jnp.concatenate(
        [ypred_ref[:, pl.ds(NT_FULL * 128, C - NT_FULL * 128)],
         jnp.zeros((R, 128 - (C - NT_FULL * 128)), jnp.float32)], axis=1)
    g = jnp.take_along_axis(xt, lk, axis=1)
    yp = jnp.where(tk == NT_FULL, g, acc)[:, :P]

    mn = jnp.maximum(jnp.max(-yp, axis=1, keepdims=True), 0.0)
    pos_loss = mn + jnp.log(
        jnp.sum(jnp.exp(-yp - mn), axis=1, keepdims=True) + jnp.exp(-mn))
    mq = jnp.max(yp, axis=1, keepdims=True)
    lse_pos = mq + jnp.log(jnp.sum(jnp.exp(yp - mq), axis=1, keepdims=True))
    aux = jnp.clip(1.0 - jnp.exp(lse_pos - all_loss), EPS, 1.0)
    neg_loss = all_loss + jnp.log(aux)
    out_ref[...] = pos_loss + neg_loss


_loss = pl.pallas_call(
    _loss_body,
    grid=(B // R,),
    in_specs=[
        pl.BlockSpec((R, PPAD), lambda i: (i, 0)),
        pl.BlockSpec((R, C), lambda i: (i, 0)),
    ],
    out_specs=pl.BlockSpec((R, 1), lambda i: (i, 0)),
    out_shape=jax.ShapeDtypeStruct((B, 1), jnp.float32),
)


def kernel(y_pred, y_true):
    yt = jnp.pad(y_true.astype(jnp.int32), ((0, 0), (0, PPAD - P)))
    out = _loss(yt, y_pred)
    return out.reshape(B)


# speedup vs baseline: 3.7077x; 3.7077x over previous
import jax
import jax.numpy as jnp
from jax import lax
from jax.experimental import pallas as pl

B = 1024
C = 100000
P = 50
EPS = 1e-07

R = 32
PPAD = 64
NT_FULL = C // 128
TAIL = C - NT_FULL * 128


def _loss_body(yt_ref, ypred_ref, out_ref):
    x = ypred_ref[...]
    m = jnp.max(x, axis=1, keepdims=True)
    m0 = jnp.maximum(m, 0.0)
    s = jnp.sum(jnp.exp(x - m0), axis=1, keepdims=True)
    all_loss = m0 + jnp.log(s + jnp.exp(-m0))

    yt = yt_ref[...]
    tk = yt >> 7
    lk = yt & 127

    x3 = x[:, :NT_FULL * 128].reshape(R, NT_FULL, 128)
    t_iota = lax.broadcasted_iota(jnp.int32, (R, NT_FULL, PPAD), 1)
    gt = jnp.where(t_iota == tk[:, None, :], 1.0, 0.0)
    w = lax.dot_general(x3, gt, (((1,), (1,)), ((0,), (0,))),
                        preferred_element_type=jnp.float32)
    xt = jnp.concatenate(
        [x[:, NT_FULL * 128:], jnp.zeros((R, 128 - TAIL), jnp.float32)],
        axis=1)
    w = w + xt[:, :, None] * jnp.where(tk == NT_FULL, 1.0, 0.0)[:, None, :]
    l_iota = lax.broadcasted_iota(jnp.int32, (R, 128, PPAD), 1)
    yp = jnp.sum(jnp.where(l_iota == lk[:, None, :], w, 0.0), axis=1)
    yp = yp[:, :P]

    mn = jnp.maximum(jnp.max(-yp, axis=1, keepdims=True), 0.0)
    pos_loss = mn + jnp.log(
        jnp.sum(jnp.exp(-yp - mn), axis=1, keepdims=True) + jnp.exp(-mn))
    mq = jnp.max(yp, axis=1, keepdims=True)
    lse_pos = mq + jnp.log(jnp.sum(jnp.exp(yp - mq), axis=1, keepdims=True))
    aux = jnp.clip(1.0 - jnp.exp(lse_pos - all_loss), EPS, 1.0)
    neg_loss = all_loss + jnp.log(aux)
    out_ref[...] = pos_loss + neg_loss


_loss = pl.pallas_call(
    _loss_body,
    grid=(B // R,),
    in_specs=[
        pl.BlockSpec((R, PPAD), lambda i: (i, 0)),
        pl.BlockSpec((R, C), lambda i: (i, 0)),
    ],
    out_specs=pl.BlockSpec((R, 1), lambda i: (i, 0)),
    out_shape=jax.ShapeDtypeStruct((B, 1), jnp.float32),
)


def kernel(y_pred, y_true):
    yt = jnp.pad(y_true.astype(jnp.int32), ((0, 0), (0, PPAD - P)))
    out = _loss(yt, y_pred)
    return out.reshape(B)
